# Initial kernel scaffold; baseline (speedup 1.0000x reference)
import jax
import jax.numpy as jnp
from jax.experimental import pallas as pl


def kernel(x, edge_index, W1, b1, W2, b2):
    raise NotImplementedError("write your pallas kernel here")


# speedup vs baseline: 16.8811x; 16.8811x over previous
import functools

import jax
import jax.numpy as jnp
from jax import lax
from jax.experimental import pallas as pl
from jax.experimental.pallas import tpu as pltpu
from jax.experimental.pallas import tpu_sc as plsc

N = 10000
NP = 10240
E = 320000
D = 128
H = 16
C = 40
CP = 48
NC = 2
NS = 16
NW = NC * NS
EPW = E // NW
BLK = 80
NBLK = EPW // BLK
RPT = NP // NS

_f32 = jnp.float32


def _mesh():
    return plsc.VectorSubcoreMesh(core_axis_name="c", subcore_axis_name="s")


_SC_PARAMS = pltpu.CompilerParams(
    needs_layout_passes=False, use_tc_tiling_on_sc=False
)


def _sc_degrees(src1, dst1):
    @functools.partial(
        pl.kernel,
        out_type=jax.ShapeDtypeStruct((2 * NW, NP), _f32),
        mesh=_mesh(),
        compiler_params=_SC_PARAMS,
        scratch_types=[
            pltpu.VMEM((EPW,), jnp.int32),
            pltpu.VMEM((EPW,), jnp.int32),
            pltpu.VMEM((NP,), _f32),
            pltpu.VMEM((NP,), _f32),
        ],
    )
    def k(s_hbm, d_hbm, out_hbm, sidx, didx, deg_o, deg_i):
        cid = lax.axis_index("c")
        sid = lax.axis_index("s")
        wid = cid * NS + sid
        pltpu.sync_copy(s_hbm.at[wid], sidx)
        pltpu.sync_copy(d_hbm.at[wid], didx)

        zeros16 = jnp.zeros((16,), _f32)

        @pl.loop(0, NP // 16)
        def _(i):
            deg_o[pl.ds(i * 16, 16)] = zeros16
            deg_i[pl.ds(i * 16, 16)] = zeros16

        ones16 = jnp.ones((16,), _f32)

        @pl.loop(0, EPW // 16)
        def _(i):
            plsc.addupdate_scatter(deg_o, [sidx[pl.ds(i * 16, 16)]], ones16)
            plsc.addupdate_scatter(deg_i, [didx[pl.ds(i * 16, 16)]], ones16)

        pltpu.sync_copy(deg_o, out_hbm.at[wid])
        pltpu.sync_copy(deg_i, out_hbm.at[NW + wid])

    return k(src1, dst1)


def _sc_scatter(h, src2, dst2, F):
    @functools.partial(
        pl.kernel,
        out_type=jax.ShapeDtypeStruct((NC, NP, F), _f32),
        mesh=_mesh(),
        compiler_params=_SC_PARAMS,
        scratch_types=[
            pltpu.VMEM((NBLK, BLK), jnp.int32),
            pltpu.VMEM((NBLK, BLK), jnp.int32),
            pltpu.VMEM((BLK, F), _f32),
            pltpu.VMEM((RPT, F), _f32),
            pltpu.VMEM_SHARED((NP, F), _f32),
        ],
    )
    def k(h_hbm, s_hbm, d_hbm, out_hbm, sidx, didx, rows, stage, agg):
        cid = lax.axis_index("c")
        sid = lax.axis_index("s")
        wid = cid * NS + sid
        pltpu.sync_copy(s_hbm.at[wid], sidx)
        pltpu.sync_copy(d_hbm.at[wid], didx)

        zeros16 = jnp.zeros((16,), _f32)

        @pl.loop(0, RPT)
        def _(r):
            row = stage.at[r]

            @pl.loop(0, F // 16)
            def _(v):
                row[pl.ds(v * 16, 16)] = zeros16

        pltpu.sync_copy(stage, agg.at[pl.ds(sid * RPT, RPT)])
        plsc.subcore_barrier()

        @pl.loop(0, NBLK)
        def _(b):
            pltpu.sync_copy(h_hbm.at[sidx.at[b]], rows)
            pltpu.sync_copy(rows, agg.at[didx.at[b]], add=True)

        plsc.subcore_barrier()
        pltpu.sync_copy(agg.at[pl.ds(sid * RPT, RPT)], stage)
        pltpu.sync_copy(stage, out_hbm.at[cid].at[pl.ds(sid * RPT, RPT)])

    return k(h, src2, dst2)


def _tc1(degT, x_p, W1):
    def body(deg_ref, x_ref, w_ref, hs_ref, ns_ref, nd_ref):
        deg = deg_ref[...]
        deg_o = jnp.sum(deg[:, :NW], axis=1, keepdims=True)
        deg_i = jnp.sum(deg[:, NW:], axis=1, keepdims=True)
        ns = lax.rsqrt(jnp.maximum(deg_o, 1.0))
        nd = lax.rsqrt(jnp.maximum(deg_i, 1.0))
        ns_ref[...] = ns
        nd_ref[...] = nd
        h = jnp.dot(x_ref[...], w_ref[...], preferred_element_type=_f32)
        hs_ref[...] = h * ns

    return pl.pallas_call(
        body,
        out_shape=(
            jax.ShapeDtypeStruct((NP, H), _f32),
            jax.ShapeDtypeStruct((NP, 1), _f32),
            jax.ShapeDtypeStruct((NP, 1), _f32),
        ),
    )(degT, x_p, W1)


def _tc2(agg1, nd, ns, b1_2, W2):
    def body(a_ref, nd_ref, ns_ref, b_ref, w_ref, out_ref):
        agg = a_ref[0] + a_ref[1]
        hrel = jnp.maximum(agg * nd_ref[...] + b_ref[...], 0.0)
        h2 = jnp.dot(hrel, w_ref[...], preferred_element_type=_f32)
        h2 = h2 * ns_ref[...]
        out_ref[:, :C] = h2
        out_ref[:, C:] = jnp.zeros((NP, CP - C), _f32)

    return pl.pallas_call(
        body,
        out_shape=jax.ShapeDtypeStruct((NP, CP), _f32),
    )(agg1, nd, ns, b1_2, W2)


def _tc3(agg2, nd, b2_2):
    def body(a_ref, nd_ref, b_ref, out_ref):
        z = (a_ref[0] + a_ref[1])[:, :C] * nd_ref[...] + b_ref[...]
        m = jnp.max(z, axis=-1, keepdims=True)
        e = jnp.exp(z - m)
        lse = jnp.log(jnp.sum(e, axis=-1, keepdims=True)) + m
        out_ref[...] = z - lse

    return pl.pallas_call(
        body,
        out_shape=jax.ShapeDtypeStruct((NP, C), _f32),
    )(agg2, nd, b2_2)


def kernel(x, edge_index, W1, b1, W2, b2):
    x_p = jnp.pad(x, ((0, NP - N), (0, 0)))
    src = edge_index[0]
    dst = edge_index[1]
    src1 = src.reshape(NW, EPW)
    dst1 = dst.reshape(NW, EPW)
    src2 = src.reshape(NW, NBLK, BLK)
    dst2 = dst.reshape(NW, NBLK, BLK)
    b1_2 = b1[None, :]
    b2_2 = b2[None, :]

    deg_parts = _sc_degrees(src1, dst1)
    degT = deg_parts.T
    h1s, ns, nd = _tc1(degT, x_p, W1)
    agg1 = _sc_scatter(h1s, src2, dst2, H)
    h2s = _tc2(agg1, nd, ns, b1_2, W2)
    agg2 = _sc_scatter(h2s, src2, dst2, CP)
    out_p = _tc3(agg2, nd, b2_2)
    return out_p[:N]


# speedup vs baseline: 31.1932x; 1.8478x over previous
import functools

import jax
import jax.numpy as jnp
from jax import lax
from jax.experimental import pallas as pl
from jax.experimental.pallas import tpu as pltpu
from jax.experimental.pallas import tpu_sc as plsc

N = 10000
NP = 10240
E = 320000
D = 128
H = 16
C = 40
CP = 48
NC = 2
NS = 16
NW = NC * NS
EPW = E // NW
BLK = 80
NBLK = EPW // BLK
RPT = NP // NS

_f32 = jnp.float32


def _mesh():
    return plsc.VectorSubcoreMesh(core_axis_name="c", subcore_axis_name="s")


_SC_PARAMS = pltpu.CompilerParams(
    needs_layout_passes=False, use_tc_tiling_on_sc=False
)


def _sc_degrees(src1, dst1):
    @functools.partial(
        pl.kernel,
        out_type=jax.ShapeDtypeStruct((2 * NW, NP), _f32),
        mesh=_mesh(),
        compiler_params=_SC_PARAMS,
        scratch_types=[
            pltpu.VMEM((EPW,), jnp.int32),
            pltpu.VMEM((EPW,), jnp.int32),
            pltpu.VMEM((NP,), _f32),
            pltpu.VMEM((NP,), _f32),
        ],
    )
    def k(s_hbm, d_hbm, out_hbm, sidx, didx, deg_o, deg_i):
        cid = lax.axis_index("c")
        sid = lax.axis_index("s")
        wid = cid * NS + sid
        pltpu.sync_copy(s_hbm.at[wid], sidx)
        pltpu.sync_copy(d_hbm.at[wid], didx)

        zeros16 = jnp.zeros((16,), _f32)

        @pl.loop(0, NP // 16)
        def _(i):
            deg_o[pl.ds(i * 16, 16)] = zeros16
            deg_i[pl.ds(i * 16, 16)] = zeros16

        ones16 = jnp.ones((16,), _f32)

        @pl.loop(0, EPW // 16)
        def _(i):
            plsc.addupdate_scatter(deg_o, [sidx[pl.ds(i * 16, 16)]], ones16)
            plsc.addupdate_scatter(deg_i, [didx[pl.ds(i * 16, 16)]], ones16)

        pltpu.sync_copy(deg_o, out_hbm.at[wid])
        pltpu.sync_copy(deg_i, out_hbm.at[NW + wid])

    return k(src1, dst1)


def _sc_scatter(h, src2, dst2, F):
    NHALF = 5
    NSET = 2 * NHALF
    ROUNDS = NBLK // NSET
    TAIL = NBLK - ROUNDS * NSET

    @functools.partial(
        pl.kernel,
        out_type=jax.ShapeDtypeStruct((NC, NP, F), _f32),
        mesh=_mesh(),
        compiler_params=_SC_PARAMS,
        scratch_types=[
            pltpu.VMEM((NBLK, BLK), jnp.int32),
            pltpu.VMEM((NBLK, BLK), jnp.int32),
            pltpu.VMEM((NSET, BLK, F), _f32),
            pltpu.VMEM((RPT, F), _f32),
            pltpu.VMEM_SHARED((NP, F), _f32),
            pltpu.SemaphoreType.DMA((NSET,)),
            pltpu.SemaphoreType.DMA((NSET,)),
        ],
    )
    def k(h_hbm, s_hbm, d_hbm, out_hbm, sidx, didx, rows, stage, agg, gsem, ssem):
        cid = lax.axis_index("c")
        sid = lax.axis_index("s")
        wid = cid * NS + sid
        pltpu.sync_copy(s_hbm.at[wid], sidx)
        pltpu.sync_copy(d_hbm.at[wid], didx)

        def gather(b, p):
            pltpu.async_copy(h_hbm.at[sidx.at[b]], rows.at[p], gsem.at[p])

        def gather_wait(p):
            pltpu.make_async_copy(
                h_hbm.at[pl.ds(0, BLK)], rows.at[p], gsem.at[p]
            ).wait()

        def scatter(b, p):
            pltpu.async_copy(rows.at[p], agg.at[didx.at[b]], ssem.at[p], add=True)

        def scatter_wait(p):
            pltpu.make_async_copy(
                rows.at[p], agg.at[pl.ds(0, BLK)], ssem.at[p]
            ).wait()

        zeros16 = jnp.zeros((16,), _f32)

        @pl.loop(0, RPT)
        def _(r):
            row = stage.at[r]

            @pl.loop(0, F // 16)
            def _(v):
                row[pl.ds(v * 16, 16)] = zeros16

        pltpu.sync_copy(stage, agg.at[pl.ds(sid * RPT, RPT)])
        plsc.subcore_barrier()

        for p in range(NSET):
            gather(p, p)

        @pl.loop(0, ROUNDS)
        def _(g):
            base = g * NSET
            for p in range(NHALF):
                gather_wait(p)
                scatter(base + p, p)
            for p in range(NHALF, NSET):
                gather_wait(p)
                scatter(base + p, p)
            for p in range(NHALF):
                nb = base + NSET + p
                scatter_wait(p)

                @pl.when(nb < NBLK)
                def _():
                    gather(nb, p)

            for p in range(NHALF, NSET):
                nb = base + NSET + p
                scatter_wait(p)

                @pl.when(nb < NBLK)
                def _():
                    gather(nb, p)

        for p in range(TAIL):
            gather_wait(p)
            scatter(ROUNDS * NSET + p, p)
        for p in range(TAIL):
            scatter_wait(p)

        plsc.subcore_barrier()
        pltpu.sync_copy(agg.at[pl.ds(sid * RPT, RPT)], stage)
        pltpu.sync_copy(stage, out_hbm.at[cid].at[pl.ds(sid * RPT, RPT)])

    return k(h, src2, dst2)


def _tc1(degT, x_p, W1):
    def body(deg_ref, x_ref, w_ref, hs_ref, ns_ref, nd_ref):
        deg = deg_ref[...]
        deg_o = jnp.sum(deg[:, :NW], axis=1, keepdims=True)
        deg_i = jnp.sum(deg[:, NW:], axis=1, keepdims=True)
        ns = lax.rsqrt(jnp.maximum(deg_o, 1.0))
        nd = lax.rsqrt(jnp.maximum(deg_i, 1.0))
        ns_ref[...] = ns
        nd_ref[...] = nd
        h = jnp.dot(x_ref[...], w_ref[...], preferred_element_type=_f32)
        hs_ref[...] = h * ns

    return pl.pallas_call(
        body,
        out_shape=(
            jax.ShapeDtypeStruct((NP, H), _f32),
            jax.ShapeDtypeStruct((NP, 1), _f32),
            jax.ShapeDtypeStruct((NP, 1), _f32),
        ),
    )(degT, x_p, W1)


def _tc2(agg1, nd, ns, b1_2, W2):
    def body(a_ref, nd_ref, ns_ref, b_ref, w_ref, out_ref):
        agg = a_ref[0] + a_ref[1]
        hrel = jnp.maximum(agg * nd_ref[...] + b_ref[...], 0.0)
        h2 = jnp.dot(hrel, w_ref[...], preferred_element_type=_f32)
        h2 = h2 * ns_ref[...]
        out_ref[:, :C] = h2
        out_ref[:, C:] = jnp.zeros((NP, CP - C), _f32)

    return pl.pallas_call(
        body,
        out_shape=jax.ShapeDtypeStruct((NP, CP), _f32),
    )(agg1, nd, ns, b1_2, W2)


def _tc3(agg2, nd, b2_2):
    def body(a_ref, nd_ref, b_ref, out_ref):
        z = (a_ref[0] + a_ref[1])[:, :C] * nd_ref[...] + b_ref[...]
        m = jnp.max(z, axis=-1, keepdims=True)
        e = jnp.exp(z - m)
        lse = jnp.log(jnp.sum(e, axis=-1, keepdims=True)) + m
        out_ref[...] = z - lse

    return pl.pallas_call(
        body,
        out_shape=jax.ShapeDtypeStruct((NP, C), _f32),
    )(agg2, nd, b2_2)


def kernel(x, edge_index, W1, b1, W2, b2):
    x_p = jnp.pad(x, ((0, NP - N), (0, 0)))
    src = edge_index[0]
    dst = edge_index[1]
    src1 = src.reshape(NW, EPW)
    dst1 = dst.reshape(NW, EPW)
    src2 = src.reshape(NW, NBLK, BLK)
    dst2 = dst.reshape(NW, NBLK, BLK)
    b1_2 = b1[None, :]
    b2_2 = b2[None, :]

    deg_parts = _sc_degrees(src1, dst1)
    degT = deg_parts.T
    h1s, ns, nd = _tc1(degT, x_p, W1)
    agg1 = _sc_scatter(h1s, src2, dst2, H)
    h2s = _tc2(agg1, nd, ns, b1_2, W2)
    agg2 = _sc_scatter(h2s, src2, dst2, CP)
    out_p = _tc3(agg2, nd, b2_2)
    return out_p[:N]


# speedup vs baseline: 32.9424x; 1.0561x over previous
import functools

import jax
import jax.numpy as jnp
from jax import lax
from jax.experimental import pallas as pl
from jax.experimental.pallas import tpu as pltpu
from jax.experimental.pallas import tpu_sc as plsc

N = 10000
NP = 10240
E = 320000
D = 128
H = 16
C = 40
CP = 48
NC = 2
NS = 16
NW = NC * NS
EPW = E // NW
BLK = 80
NBLK = EPW // BLK
RPT = NP // NS

_f32 = jnp.float32


def _mesh():
    return plsc.VectorSubcoreMesh(core_axis_name="c", subcore_axis_name="s")


_SC_PARAMS = pltpu.CompilerParams(
    needs_layout_passes=False, use_tc_tiling_on_sc=False
)


def _sc_degrees(edge_index):
    @functools.partial(
        pl.kernel,
        out_type=jax.ShapeDtypeStruct((2 * NW, NP), _f32),
        mesh=_mesh(),
        compiler_params=_SC_PARAMS,
        scratch_types=[
            pltpu.VMEM((EPW,), jnp.int32),
            pltpu.VMEM((EPW,), jnp.int32),
            pltpu.VMEM((NP,), _f32),
            pltpu.VMEM((NP,), _f32),
        ],
    )
    def k(e_hbm, out_hbm, sidx, didx, deg_o, deg_i):
        cid = lax.axis_index("c")
        sid = lax.axis_index("s")
        wid = cid * NS + sid
        pltpu.sync_copy(e_hbm.at[0].at[pl.ds(wid * EPW, EPW)], sidx)
        pltpu.sync_copy(e_hbm.at[1].at[pl.ds(wid * EPW, EPW)], didx)

        zeros16 = jnp.zeros((16,), _f32)

        @pl.loop(0, NP // 16)
        def _(i):
            deg_o[pl.ds(i * 16, 16)] = zeros16
            deg_i[pl.ds(i * 16, 16)] = zeros16

        ones16 = jnp.ones((16,), _f32)

        @pl.loop(0, EPW // 16)
        def _(i):
            plsc.addupdate_scatter(deg_o, [sidx[pl.ds(i * 16, 16)]], ones16)
            plsc.addupdate_scatter(deg_i, [didx[pl.ds(i * 16, 16)]], ones16)

        pltpu.sync_copy(deg_o, out_hbm.at[wid])
        pltpu.sync_copy(deg_i, out_hbm.at[NW + wid])

    return k(edge_index)


def _sc_scatter(h, edge_index, F):
    NHALF = 5
    NSET = 2 * NHALF
    ROUNDS = NBLK // NSET
    TAIL = NBLK - ROUNDS * NSET

    @functools.partial(
        pl.kernel,
        out_type=jax.ShapeDtypeStruct((NC, NP, F), _f32),
        mesh=_mesh(),
        compiler_params=_SC_PARAMS,
        scratch_types=[
            pltpu.VMEM((EPW,), jnp.int32),
            pltpu.VMEM((EPW,), jnp.int32),
            pltpu.VMEM((NSET, BLK, F), _f32),
            pltpu.VMEM((RPT, F), _f32),
            pltpu.VMEM_SHARED((NP, F), _f32),
            pltpu.SemaphoreType.DMA((NSET,)),
            pltpu.SemaphoreType.DMA((NSET,)),
        ],
    )
    def k(h_hbm, e_hbm, out_hbm, sidx, didx, rows, stage, agg, gsem, ssem):
        cid = lax.axis_index("c")
        sid = lax.axis_index("s")
        wid = cid * NS + sid
        pltpu.sync_copy(e_hbm.at[0].at[pl.ds(wid * EPW, EPW)], sidx)
        pltpu.sync_copy(e_hbm.at[1].at[pl.ds(wid * EPW, EPW)], didx)

        def gather(b, p):
            pltpu.async_copy(
                h_hbm.at[sidx.at[pl.ds(b * BLK, BLK)]], rows.at[p], gsem.at[p]
            )

        def gather_wait(p):
            pltpu.make_async_copy(
                h_hbm.at[pl.ds(0, BLK)], rows.at[p], gsem.at[p]
            ).wait()

        def scatter(b, p):
            pltpu.async_copy(
                rows.at[p], agg.at[didx.at[pl.ds(b * BLK, BLK)]], ssem.at[p],
                add=True,
            )

        def scatter_wait(p):
            pltpu.make_async_copy(
                rows.at[p], agg.at[pl.ds(0, BLK)], ssem.at[p]
            ).wait()

        zeros16 = jnp.zeros((16,), _f32)

        @pl.loop(0, RPT)
        def _(r):
            row = stage.at[r]

            @pl.loop(0, F // 16)
            def _(v):
                row[pl.ds(v * 16, 16)] = zeros16

        pltpu.sync_copy(stage, agg.at[pl.ds(sid * RPT, RPT)])
        plsc.subcore_barrier()

        for p in range(NSET):
            gather(p, p)

        @pl.loop(0, ROUNDS)
        def _(g):
            base = g * NSET
            for p in range(NHALF):
                gather_wait(p)
                scatter(base + p, p)
            for p in range(NHALF, NSET):
                gather_wait(p)
                scatter(base + p, p)
            for p in range(NHALF):
                nb = base + NSET + p
                scatter_wait(p)

                @pl.when(nb < NBLK)
                def _():
                    gather(nb, p)

            for p in range(NHALF, NSET):
                nb = base + NSET + p
                scatter_wait(p)

                @pl.when(nb < NBLK)
                def _():
                    gather(nb, p)

        for p in range(TAIL):
            gather_wait(p)
            scatter(ROUNDS * NSET + p, p)
        for p in range(TAIL):
            scatter_wait(p)

        plsc.subcore_barrier()
        pltpu.sync_copy(agg.at[pl.ds(sid * RPT, RPT)], stage)
        pltpu.sync_copy(stage, out_hbm.at[cid].at[pl.ds(sid * RPT, RPT)])

    return k(h, edge_index)


def _tc0(x_p, W1):
    def body(x_ref, w_ref, h_ref):
        h_ref[...] = jnp.dot(x_ref[...], w_ref[...], preferred_element_type=_f32)

    return pl.pallas_call(
        body,
        out_shape=jax.ShapeDtypeStruct((NP, H), _f32),
    )(x_p, W1)


def _tc1(degT, h1):
    def body(deg_ref, h_ref, hs_ref, ns_ref, nd_ref):
        deg = deg_ref[...]
        deg_o = jnp.sum(deg[:, :NW], axis=1, keepdims=True)
        deg_i = jnp.sum(deg[:, NW:], axis=1, keepdims=True)
        ns = lax.rsqrt(jnp.maximum(deg_o, 1.0))
        nd = lax.rsqrt(jnp.maximum(deg_i, 1.0))
        ns_ref[...] = ns
        nd_ref[...] = nd
        hs_ref[...] = h_ref[...] * ns

    return pl.pallas_call(
        body,
        out_shape=(
            jax.ShapeDtypeStruct((NP, H), _f32),
            jax.ShapeDtypeStruct((NP, 1), _f32),
            jax.ShapeDtypeStruct((NP, 1), _f32),
        ),
    )(degT, h1)


def _tc2(agg1, nd, ns, b1_2, W2):
    def body(a_ref, nd_ref, ns_ref, b_ref, w_ref, out_ref):
        agg = a_ref[0] + a_ref[1]
        hrel = jnp.maximum(agg * nd_ref[...] + b_ref[...], 0.0)
        h2 = jnp.dot(hrel, w_ref[...], preferred_element_type=_f32)
        h2 = h2 * ns_ref[...]
        out_ref[:, :C] = h2
        out_ref[:, C:] = jnp.zeros((NP, CP - C), _f32)

    return pl.pallas_call(
        body,
        out_shape=jax.ShapeDtypeStruct((NP, CP), _f32),
    )(agg1, nd, ns, b1_2, W2)


def _tc3(agg2, nd, b2_2):
    def body(a_ref, nd_ref, b_ref, out_ref):
        z = (a_ref[0] + a_ref[1])[:, :C] * nd_ref[...] + b_ref[...]
        m = jnp.max(z, axis=-1, keepdims=True)
        e = jnp.exp(z - m)
        lse = jnp.log(jnp.sum(e, axis=-1, keepdims=True)) + m
        out_ref[...] = z - lse

    return pl.pallas_call(
        body,
        out_shape=jax.ShapeDtypeStruct((NP, C), _f32),
    )(agg2, nd, b2_2)


def kernel(x, edge_index, W1, b1, W2, b2):
    x_p = jnp.pad(x, ((0, NP - N), (0, 0)))
    b1_2 = b1[None, :]
    b2_2 = b2[None, :]

    deg_parts = _sc_degrees(edge_index)
    h1 = _tc0(x_p, W1)
    degT = deg_parts.T
    h1s, ns, nd = _tc1(degT, h1)
    agg1 = _sc_scatter(h1s, edge_index, H)
    h2s = _tc2(agg1, nd, ns, b1_2, W2)
    agg2 = _sc_scatter(h2s, edge_index, CP)
    out_p = _tc3(agg2, nd, b2_2)
    return out_p[:N]
